# baseline (speedup 1.0000x reference)
import jax
import jax.numpy as jnp
import numpy as np
from jax import lax
from jax.experimental import pallas as pl
from jax.experimental.pallas import tpu as pltpu
from jax.experimental.pallas import tpu_sc as plsc

_F32 = jnp.float32
_BF16 = jnp.bfloat16
_I32 = jnp.int32

_NC = 2
_NS = 16
_NW = _NC * _NS
_LANES = 16


def _lo(w):
    return lax.bitcast_convert_type(jnp.left_shift(w, 16), _F32)


def _hi(w):
    return lax.bitcast_convert_type(jnp.bitwise_and(w, _I32(-65536)), _F32)


def _make_esum(rows, n, h_dim, chunk_rows):
    words_per_j = h_dim // 2
    vecs_per_j = words_per_j // _LANES
    words_per_row = n * words_per_j
    rows_per_w = rows // _NW
    n_chunks = rows_per_w // chunk_rows
    chunk_words = chunk_rows * words_per_row
    assert vecs_per_j == 2 and rows_per_w * _NW == rows
    assert n_chunks * chunk_rows == rows_per_w and n % 4 == 0

    def body(e_hbm, out_hbm, buf0, buf1, outbuf, sem0, sem1):
        wid = lax.axis_index("s") * _NC + lax.axis_index("c")
        base = wid * (rows_per_w * words_per_row)
        bufs = (buf0, buf1)
        sems = (sem0, sem1)

        def start(c):
            return pltpu.async_copy(
                e_hbm.at[pl.ds(base + c * chunk_words, chunk_words)],
                bufs[c % 2], sems[c % 2])

        cp = start(0)
        for c in range(n_chunks):
            nxt = start(c + 1) if c + 1 < n_chunks else None
            cp.wait()
            buf = bufs[c % 2]
            for r in range(chunk_rows):
                rb = r * words_per_row

                def jbody(j0, accs, buf=buf, rb=rb):
                    a0, a1, a2, a3 = accs
                    jb = rb + j0 * (4 * words_per_j)
                    p0, p1, p2, p3 = [], [], [], []
                    for u in range(4):
                        o = jb + u * words_per_j
                        va = buf[pl.ds(o, _LANES)]
                        vb = buf[pl.ds(o + _LANES, _LANES)]
                        p0.append(_lo(va))
                        p1.append(_hi(va))
                        p2.append(_lo(vb))
                        p3.append(_hi(vb))
                    tree = lambda p: (p[0] + p[1]) + (p[2] + p[3])
                    return (a0 + tree(p0), a1 + tree(p1),
                            a2 + tree(p2), a3 + tree(p3))

                z = jnp.zeros((_LANES,), _F32)
                accs = lax.fori_loop(0, n // 4, jbody, (z, z, z, z))
                ob = (c * chunk_rows + r) * h_dim
                for k in range(4):
                    outbuf[pl.ds(ob + k * _LANES, _LANES)] = accs[k]
            cp = nxt
        out_words = rows_per_w * h_dim
        pltpu.sync_copy(outbuf, out_hbm.at[pl.ds(wid * out_words, out_words)])

    return pl.kernel(
        body,
        mesh=plsc.VectorSubcoreMesh(core_axis_name="c", subcore_axis_name="s"),
        out_type=jax.ShapeDtypeStruct((rows * h_dim,), _F32),
        scratch_types=[
            pltpu.VMEM((chunk_words,), _I32),
            pltpu.VMEM((chunk_words,), _I32),
            pltpu.VMEM((rows_per_w * h_dim,), _F32),
            pltpu.SemaphoreType.DMA,
            pltpu.SemaphoreType.DMA,
        ],
    )


def _dot(a, b, precision=None):
    return lax.dot_general(
        a, b, (((1,), (0,)), ((), ())),
        precision=precision,
        preferred_element_type=_F32,
    )


def _gru_kernel(y_ref, node_ref, wfab_ref, wfc_ref, bf_ref, wih_ref,
                whh_ref, bih_ref, bhh_ref, out_ref):
    n = node_ref.shape[1]
    h_dim = node_ref.shape[2]
    r_rounds = wfab_ref.shape[0]
    hi = lax.Precision.HIGHEST

    y = y_ref[0]
    h = node_ref[0]
    fn = _F32(n)
    for t in range(r_rounds):
        h16 = h.astype(_BF16)
        hterm = _dot(h16, wfab_ref[t, :h_dim, :]) * fn
        hs = jnp.sum(h16.astype(_F32), axis=0, keepdims=True)
        hsterm = _dot(hs, wfab_ref[t, h_dim:, :].astype(_F32), hi)
        eterm = _dot(y, wfc_ref[t].astype(_F32), hi)
        agg = (hterm + jnp.broadcast_to(hsterm, (n, 6 * h_dim)) + eterm
               + fn * bf_ref[t][None, :])
        gi = _dot(agg.astype(_BF16), wih_ref[t]) + bih_ref[t][None, :]
        gh = _dot(h16, whh_ref[t]) + bhh_ref[t][None, :]
        i_r, i_z, i_n = jnp.split(gi, 3, axis=-1)
        h_r, h_z, h_n = jnp.split(gh, 3, axis=-1)
        r = jax.nn.sigmoid(i_r + h_r)
        z = jax.nn.sigmoid(i_z + h_z)
        nn = jnp.tanh(i_n + r * h_n)
        h = (1.0 - z) * nn + z * h

    out_ref[...] = jnp.sum(h, axis=0, keepdims=True)[None]


def kernel(embedded_node, embedded_adjancy_matrix, Wf, bf, Wih, Whh, bih, bhh):
    b_g, n, _, h_dim = embedded_adjancy_matrix.shape
    r_rounds = Wf.shape[0]
    rows = b_g * n

    e16 = embedded_adjancy_matrix.astype(_BF16)
    e_words = lax.bitcast_convert_type(e16.reshape(-1, 2), _I32)

    y = _make_esum(rows, n, h_dim, chunk_rows=4)(e_words)
    y = y.reshape(b_g, n, h_dim)

    half = h_dim // 2
    perm = np.concatenate([
        np.arange(0, half, 2), np.arange(1, half, 2),
        np.arange(half, h_dim, 2), np.arange(half + 1, h_dim, 2)])
    wfc16 = Wf[:, 2 * h_dim:, :][:, perm, :].astype(_BF16)
    wfab16 = Wf[:, :2 * h_dim, :].astype(_BF16)
    wih16 = Wih.astype(_BF16)
    whh16 = Whh.astype(_BF16)

    out = pl.pallas_call(
        _gru_kernel,
        grid=(b_g,),
        in_specs=[
            pl.BlockSpec((1, n, h_dim), lambda b: (b, 0, 0)),
            pl.BlockSpec((1, n, h_dim), lambda b: (b, 0, 0)),
            pl.BlockSpec((r_rounds, 2 * h_dim, 6 * h_dim), lambda b: (0, 0, 0)),
            pl.BlockSpec((r_rounds, h_dim, 6 * h_dim), lambda b: (0, 0, 0)),
            pl.BlockSpec((r_rounds, 6 * h_dim), lambda b: (0, 0)),
            pl.BlockSpec((r_rounds, 6 * h_dim, 3 * h_dim), lambda b: (0, 0, 0)),
            pl.BlockSpec((r_rounds, h_dim, 3 * h_dim), lambda b: (0, 0, 0)),
            pl.BlockSpec((r_rounds, 3 * h_dim), lambda b: (0, 0)),
            pl.BlockSpec((r_rounds, 3 * h_dim), lambda b: (0, 0)),
        ],
        out_specs=pl.BlockSpec((1, 1, h_dim), lambda b: (b, 0, 0)),
        out_shape=jax.ShapeDtypeStruct((b_g, 1, h_dim), jnp.float32),
        compiler_params=pltpu.CompilerParams(
            dimension_semantics=("arbitrary",),
        ),
    )(y, embedded_node, wfab16, wfc16, bf, wih16, whh16, bih, bhh)
    return out.reshape(b_g, h_dim)


# speedup vs baseline: 16.9879x; 16.9879x over previous
import jax
import jax.numpy as jnp
from jax import lax
from jax.experimental import pallas as pl
from jax.experimental.pallas import tpu as pltpu
from jax.experimental.pallas import tpu_sc as plsc

_F32 = jnp.float32
_BF16 = jnp.bfloat16
_I32 = jnp.int32

_NC = 2
_NS = 16
_NW = _NC * _NS
_LANES = 16


def _round_bf16(v):
    u = lax.bitcast_convert_type(v, _I32)
    lsb = jnp.bitwise_and(jnp.right_shift(u, 16), 1)
    r = jnp.bitwise_and(u + 32767 + lsb, _I32(-65536))
    return lax.bitcast_convert_type(r, _F32)


def _make_esum(rows, n, h_dim, chunk_rows):
    vecs_per_j = h_dim // _LANES
    words_per_row = n * h_dim
    rows_per_w = rows // _NW
    n_chunks = rows_per_w // chunk_rows
    chunk_words = chunk_rows * words_per_row
    assert vecs_per_j == 4 and rows_per_w * _NW == rows
    assert n_chunks * chunk_rows == rows_per_w and n % 4 == 0

    def body(e_hbm, out_hbm, buf0, buf1, outbuf, sem0, sem1):
        wid = lax.axis_index("s") * _NC + lax.axis_index("c")
        base = wid * (rows_per_w * words_per_row)
        bufs = (buf0, buf1)
        sems = (sem0, sem1)

        def start(c):
            return pltpu.async_copy(
                e_hbm.at[pl.ds(base + c * chunk_words, chunk_words)],
                bufs[c % 2], sems[c % 2])

        cp = start(0)
        for c in range(n_chunks):
            nxt = start(c + 1) if c + 1 < n_chunks else None
            cp.wait()
            buf = bufs[c % 2]
            for r in range(chunk_rows):
                rb = r * words_per_row

                def jbody(j0, accs, buf=buf, rb=rb):
                    a0, a1, a2, a3 = accs
                    jb = rb + j0 * (4 * h_dim)
                    p0, p1, p2, p3 = [], [], [], []
                    for u in range(4):
                        o = jb + u * h_dim
                        p0.append(_round_bf16(buf[pl.ds(o, _LANES)]))
                        p1.append(_round_bf16(buf[pl.ds(o + _LANES, _LANES)]))
                        p2.append(_round_bf16(buf[pl.ds(o + 2 * _LANES, _LANES)]))
                        p3.append(_round_bf16(buf[pl.ds(o + 3 * _LANES, _LANES)]))
                    tree = lambda p: (p[0] + p[1]) + (p[2] + p[3])
                    return (a0 + tree(p0), a1 + tree(p1),
                            a2 + tree(p2), a3 + tree(p3))

                z = jnp.zeros((_LANES,), _F32)
                accs = lax.fori_loop(0, n // 4, jbody, (z, z, z, z))
                ob = (c * chunk_rows + r) * h_dim
                for k in range(4):
                    outbuf[pl.ds(ob + k * _LANES, _LANES)] = accs[k]
            cp = nxt
        out_words = rows_per_w * h_dim
        pltpu.sync_copy(outbuf, out_hbm.at[pl.ds(wid * out_words, out_words)])

    return pl.kernel(
        body,
        mesh=plsc.VectorSubcoreMesh(core_axis_name="c", subcore_axis_name="s"),
        out_type=jax.ShapeDtypeStruct((rows * h_dim,), _F32),
        scratch_types=[
            pltpu.VMEM((chunk_words,), _F32),
            pltpu.VMEM((chunk_words,), _F32),
            pltpu.VMEM((rows_per_w * h_dim,), _F32),
            pltpu.SemaphoreType.DMA,
            pltpu.SemaphoreType.DMA,
        ],
    )


def _dot(a, b, precision=None):
    return lax.dot_general(
        a, b, (((1,), (0,)), ((), ())),
        precision=precision,
        preferred_element_type=_F32,
    )


def _gru_kernel(y_ref, node_ref, wfab_ref, wfc_ref, bf_ref, wih_ref,
                whh_ref, bih_ref, bhh_ref, out_ref):
    n = node_ref.shape[1]
    h_dim = node_ref.shape[2]
    r_rounds = wfab_ref.shape[0]
    hi = lax.Precision.HIGHEST

    y = y_ref[0]
    h = node_ref[0]
    fn = _F32(n)
    for t in range(r_rounds):
        h16 = h.astype(_BF16)
        hterm = _dot(h16, wfab_ref[t, :h_dim, :]) * fn
        hs = jnp.sum(h16.astype(_F32), axis=0, keepdims=True)
        hsterm = _dot(hs, wfab_ref[t, h_dim:, :].astype(_F32), hi)
        eterm = _dot(y, wfc_ref[t].astype(_F32), hi)
        agg = (hterm + jnp.broadcast_to(hsterm, (n, 6 * h_dim)) + eterm
               + fn * bf_ref[t][None, :])
        gi = _dot(agg.astype(_BF16), wih_ref[t]) + bih_ref[t][None, :]
        gh = _dot(h16, whh_ref[t]) + bhh_ref[t][None, :]
        i_r, i_z, i_n = jnp.split(gi, 3, axis=-1)
        h_r, h_z, h_n = jnp.split(gh, 3, axis=-1)
        r = jax.nn.sigmoid(i_r + h_r)
        z = jax.nn.sigmoid(i_z + h_z)
        nn = jnp.tanh(i_n + r * h_n)
        h = (1.0 - z) * nn + z * h

    out_ref[...] = jnp.sum(h, axis=0, keepdims=True)[None]


def kernel(embedded_node, embedded_adjancy_matrix, Wf, bf, Wih, Whh, bih, bhh):
    b_g, n, _, h_dim = embedded_adjancy_matrix.shape
    r_rounds = Wf.shape[0]
    rows = b_g * n

    e_flat = embedded_adjancy_matrix.reshape(-1)
    y = _make_esum(rows, n, h_dim, chunk_rows=4)(e_flat)
    y = y.reshape(b_g, n, h_dim)

    wfc16 = Wf[:, 2 * h_dim:, :].astype(_BF16)
    wfab16 = Wf[:, :2 * h_dim, :].astype(_BF16)
    wih16 = Wih.astype(_BF16)
    whh16 = Whh.astype(_BF16)

    out = pl.pallas_call(
        _gru_kernel,
        grid=(b_g,),
        in_specs=[
            pl.BlockSpec((1, n, h_dim), lambda b: (b, 0, 0)),
            pl.BlockSpec((1, n, h_dim), lambda b: (b, 0, 0)),
            pl.BlockSpec((r_rounds, 2 * h_dim, 6 * h_dim), lambda b: (0, 0, 0)),
            pl.BlockSpec((r_rounds, h_dim, 6 * h_dim), lambda b: (0, 0, 0)),
            pl.BlockSpec((r_rounds, 6 * h_dim), lambda b: (0, 0)),
            pl.BlockSpec((r_rounds, 6 * h_dim, 3 * h_dim), lambda b: (0, 0, 0)),
            pl.BlockSpec((r_rounds, h_dim, 3 * h_dim), lambda b: (0, 0, 0)),
            pl.BlockSpec((r_rounds, 3 * h_dim), lambda b: (0, 0)),
            pl.BlockSpec((r_rounds, 3 * h_dim), lambda b: (0, 0)),
        ],
        out_specs=pl.BlockSpec((1, 1, h_dim), lambda b: (b, 0, 0)),
        out_shape=jax.ShapeDtypeStruct((b_g, 1, h_dim), jnp.float32),
        compiler_params=pltpu.CompilerParams(
            dimension_semantics=("arbitrary",),
        ),
    )(y, embedded_node, wfab16, wfc16, bf, wih16, whh16, bih, bhh)
    return out.reshape(b_g, h_dim)


# speedup vs baseline: 20.8168x; 1.2254x over previous
import jax
import jax.numpy as jnp
from jax import lax
from jax.experimental import pallas as pl
from jax.experimental.pallas import tpu as pltpu
from jax.experimental.pallas import tpu_sc as plsc

_F32 = jnp.float32
_BF16 = jnp.bfloat16
_I32 = jnp.int32

_NC = 2
_NS = 16
_NW = _NC * _NS
_LANES = 16


def _round_bf16(v):
    u = lax.bitcast_convert_type(v, _I32)
    lsb = jnp.bitwise_and(jnp.right_shift(u, 16), 1)
    r = jnp.bitwise_and(u + 32767 + lsb, _I32(-65536))
    return lax.bitcast_convert_type(r, _F32)


def _make_esum(b_g, n, h_dim, chunk_rows):
    rows = b_g * n
    rows_per_w = rows // _NW
    i_per_w = rows_per_w
    n_chunks = rows_per_w // chunk_rows
    assert h_dim == 4 * _LANES and rows_per_w * _NW == rows
    assert n % rows_per_w == 0
    assert n_chunks * chunk_rows == rows_per_w and n % 4 == 0

    def body(e_hbm, out_hbm, buf0, buf1, outbuf, sem0, sem1):
        wid = lax.axis_index("s") * _NC + lax.axis_index("c")
        b = wid // (n // i_per_w)
        i0 = (wid % (n // i_per_w)) * i_per_w
        bufs = (buf0, buf1)
        sems = (sem0, sem1)

        def start(c):
            return pltpu.async_copy(
                e_hbm.at[b, pl.ds(i0 + c * chunk_rows, chunk_rows)],
                bufs[c % 2], sems[c % 2])

        cp = start(0)
        for c in range(n_chunks):
            nxt = start(c + 1) if c + 1 < n_chunks else None
            cp.wait()
            buf = bufs[c % 2]
            for r in range(chunk_rows):

                def jbody(j0, accs, buf=buf, r=r):
                    a0, a1, a2, a3 = accs
                    j = j0 * 4
                    p0, p1, p2, p3 = [], [], [], []
                    for u in range(4):
                        p0.append(_round_bf16(buf[r, j + u, pl.ds(0, _LANES)]))
                        p1.append(_round_bf16(buf[r, j + u, pl.ds(_LANES, _LANES)]))
                        p2.append(_round_bf16(buf[r, j + u, pl.ds(2 * _LANES, _LANES)]))
                        p3.append(_round_bf16(buf[r, j + u, pl.ds(3 * _LANES, _LANES)]))
                    tree = lambda p: (p[0] + p[1]) + (p[2] + p[3])
                    return (a0 + tree(p0), a1 + tree(p1),
                            a2 + tree(p2), a3 + tree(p3))

                z = jnp.zeros((_LANES,), _F32)
                accs = lax.fori_loop(0, n // 4, jbody, (z, z, z, z))
                ob = (c * chunk_rows + r) * h_dim
                for k in range(4):
                    outbuf[pl.ds(ob + k * _LANES, _LANES)] = accs[k]
            cp = nxt
        out_words = rows_per_w * h_dim
        pltpu.sync_copy(outbuf, out_hbm.at[pl.ds(wid * out_words, out_words)])

    return pl.kernel(
        body,
        mesh=plsc.VectorSubcoreMesh(core_axis_name="c", subcore_axis_name="s"),
        out_type=jax.ShapeDtypeStruct((rows * h_dim,), _F32),
        scratch_types=[
            pltpu.VMEM((chunk_rows, n, h_dim), _F32),
            pltpu.VMEM((chunk_rows, n, h_dim), _F32),
            pltpu.VMEM((rows_per_w * h_dim,), _F32),
            pltpu.SemaphoreType.DMA,
            pltpu.SemaphoreType.DMA,
        ],
    )


def _dot(a, b, precision=None):
    return lax.dot_general(
        a, b, (((1,), (0,)), ((), ())),
        precision=precision,
        preferred_element_type=_F32,
    )


def _gru_kernel(y_ref, node_ref, wfab_ref, wfc_ref, bf_ref, wih_ref,
                whh_ref, bih_ref, bhh_ref, out_ref):
    n = node_ref.shape[1]
    h_dim = node_ref.shape[2]
    r_rounds = wfab_ref.shape[0]
    hi = lax.Precision.HIGHEST

    y = y_ref[0]
    h = node_ref[0]
    fn = _F32(n)
    for t in range(r_rounds):
        h16 = h.astype(_BF16)
        hterm = _dot(h16, wfab_ref[t, :h_dim, :]) * fn
        hs = jnp.sum(h16.astype(_F32), axis=0, keepdims=True)
        hsterm = _dot(hs, wfab_ref[t, h_dim:, :].astype(_F32), hi)
        eterm = _dot(y, wfc_ref[t].astype(_F32), hi)
        agg = (hterm + jnp.broadcast_to(hsterm, (n, 6 * h_dim)) + eterm
               + fn * bf_ref[t][None, :])
        gi = _dot(agg.astype(_BF16), wih_ref[t]) + bih_ref[t][None, :]
        gh = _dot(h16, whh_ref[t]) + bhh_ref[t][None, :]
        i_r, i_z, i_n = jnp.split(gi, 3, axis=-1)
        h_r, h_z, h_n = jnp.split(gh, 3, axis=-1)
        r = jax.nn.sigmoid(i_r + h_r)
        z = jax.nn.sigmoid(i_z + h_z)
        nn = jnp.tanh(i_n + r * h_n)
        h = (1.0 - z) * nn + z * h

    out_ref[...] = jnp.sum(h, axis=0, keepdims=True)[None]


def kernel(embedded_node, embedded_adjancy_matrix, Wf, bf, Wih, Whh, bih, bhh):
    b_g, n, _, h_dim = embedded_adjancy_matrix.shape
    r_rounds = Wf.shape[0]
    rows = b_g * n

    y = _make_esum(b_g, n, h_dim, chunk_rows=2)(embedded_adjancy_matrix)
    y = y.reshape(b_g, n, h_dim)

    wfc16 = Wf[:, 2 * h_dim:, :].astype(_BF16)
    wfab16 = Wf[:, :2 * h_dim, :].astype(_BF16)
    wih16 = Wih.astype(_BF16)
    whh16 = Whh.astype(_BF16)

    out = pl.pallas_call(
        _gru_kernel,
        grid=(b_g,),
        in_specs=[
            pl.BlockSpec((1, n, h_dim), lambda b: (b, 0, 0)),
            pl.BlockSpec((1, n, h_dim), lambda b: (b, 0, 0)),
            pl.BlockSpec((r_rounds, 2 * h_dim, 6 * h_dim), lambda b: (0, 0, 0)),
            pl.BlockSpec((r_rounds, h_dim, 6 * h_dim), lambda b: (0, 0, 0)),
            pl.BlockSpec((r_rounds, 6 * h_dim), lambda b: (0, 0)),
            pl.BlockSpec((r_rounds, 6 * h_dim, 3 * h_dim), lambda b: (0, 0, 0)),
            pl.BlockSpec((r_rounds, h_dim, 3 * h_dim), lambda b: (0, 0, 0)),
            pl.BlockSpec((r_rounds, 3 * h_dim), lambda b: (0, 0)),
            pl.BlockSpec((r_rounds, 3 * h_dim), lambda b: (0, 0)),
        ],
        out_specs=pl.BlockSpec((1, 1, h_dim), lambda b: (b, 0, 0)),
        out_shape=jax.ShapeDtypeStruct((b_g, 1, h_dim), jnp.float32),
        compiler_params=pltpu.CompilerParams(
            dimension_semantics=("arbitrary",),
        ),
    )(y, embedded_node, wfab16, wfc16, bf, wih16, whh16, bih, bhh)
    return out.reshape(b_g, h_dim)


# speedup vs baseline: 22.5541x; 1.0835x over previous
import jax
import jax.numpy as jnp
from jax import lax
from jax.experimental import pallas as pl
from jax.experimental.pallas import tpu as pltpu
from jax.experimental.pallas import tpu_sc as plsc

_F32 = jnp.float32
_BF16 = jnp.bfloat16
_I32 = jnp.int32

_NC = 2
_NS = 16
_NW = _NC * _NS
_LANES = 16


def _round_bf16(v):
    u = lax.bitcast_convert_type(v, _I32)
    lsb = jnp.bitwise_and(jnp.right_shift(u, 16), 1)
    r = jnp.bitwise_and(u + 32767 + lsb, _I32(-65536))
    return lax.bitcast_convert_type(r, _F32)


def _make_esum(b_g, n, h_dim, chunk_rows):
    rows = b_g * n
    rows_per_w = rows // _NW
    n_chunks = rows_per_w // chunk_rows
    jvecs = n // _LANES
    assert h_dim == 4 * _LANES and rows_per_w * _NW == rows
    assert n % rows_per_w == 0
    assert n_chunks * chunk_rows == rows_per_w and n % _LANES == 0

    def body(e_hbm, out_hbm, buf0, buf1, outbuf, sem0, sem1):
        wid = lax.axis_index("s") * _NC + lax.axis_index("c")
        b = wid // (n // rows_per_w)
        i0 = (wid % (n // rows_per_w)) * rows_per_w
        bufs = (buf0, buf1)
        sems = (sem0, sem1)

        def start(c):
            return pltpu.async_copy(
                e_hbm.at[b, pl.ds(i0 + c * chunk_rows, chunk_rows)],
                bufs[c % 2], sems[c % 2])

        cp = start(0)
        for c in range(n_chunks):
            nxt = start(c + 1) if c + 1 < n_chunks else None
            cp.wait()
            buf = bufs[c % 2]
            for r in range(chunk_rows):
                ob = (c * chunk_rows + r) * h_dim * _LANES

                def hbody(h, carry, buf=buf, r=r, ob=ob):
                    p = [_round_bf16(buf[r, h, pl.ds(g * _LANES, _LANES)])
                         for g in range(jvecs)]
                    while len(p) > 1:
                        p = [a + b for a, b in zip(p[::2], p[1::2])]
                    outbuf[pl.ds(ob + h * _LANES, _LANES)] = p[0]
                    return carry

                lax.fori_loop(0, h_dim, hbody, 0)
            cp = nxt
        out_words = rows_per_w * h_dim * _LANES
        pltpu.sync_copy(outbuf, out_hbm.at[pl.ds(wid * out_words, out_words)])

    return pl.kernel(
        body,
        mesh=plsc.VectorSubcoreMesh(core_axis_name="c", subcore_axis_name="s"),
        out_type=jax.ShapeDtypeStruct((rows * h_dim * _LANES,), _F32),
        scratch_types=[
            pltpu.VMEM((chunk_rows, h_dim, n), _F32),
            pltpu.VMEM((chunk_rows, h_dim, n), _F32),
            pltpu.VMEM((rows_per_w * h_dim * _LANES,), _F32),
            pltpu.SemaphoreType.DMA,
            pltpu.SemaphoreType.DMA,
        ],
    )


def _dot(a, b, precision=None):
    return lax.dot_general(
        a, b, (((1,), (0,)), ((), ())),
        precision=precision,
        preferred_element_type=_F32,
    )


def _gru_kernel(y_ref, node_ref, wfab_ref, wfc_ref, bf_ref, wih_ref,
                whh_ref, bih_ref, bhh_ref, out_ref):
    n = node_ref.shape[1]
    h_dim = node_ref.shape[2]
    r_rounds = wfab_ref.shape[0]
    hi = lax.Precision.HIGHEST

    y = y_ref[0]
    h = node_ref[0]
    fn = _F32(n)
    for t in range(r_rounds):
        h16 = h.astype(_BF16)
        hterm = _dot(h16, wfab_ref[t, :h_dim, :]) * fn
        hs = jnp.sum(h16.astype(_F32), axis=0, keepdims=True)
        hsterm = _dot(hs, wfab_ref[t, h_dim:, :].astype(_F32), hi)
        eterm = _dot(y, wfc_ref[t].astype(_F32), hi)
        agg = (hterm + jnp.broadcast_to(hsterm, (n, 6 * h_dim)) + eterm
               + fn * bf_ref[t][None, :])
        gi = _dot(agg.astype(_BF16), wih_ref[t]) + bih_ref[t][None, :]
        gh = _dot(h16, whh_ref[t]) + bhh_ref[t][None, :]
        i_r, i_z, i_n = jnp.split(gi, 3, axis=-1)
        h_r, h_z, h_n = jnp.split(gh, 3, axis=-1)
        r = jax.nn.sigmoid(i_r + h_r)
        z = jax.nn.sigmoid(i_z + h_z)
        nn = jnp.tanh(i_n + r * h_n)
        h = (1.0 - z) * nn + z * h

    out_ref[...] = jnp.sum(h, axis=0, keepdims=True)[None]


def kernel(embedded_node, embedded_adjancy_matrix, Wf, bf, Wih, Whh, bih, bhh):
    b_g, n, _, h_dim = embedded_adjancy_matrix.shape
    r_rounds = Wf.shape[0]
    rows = b_g * n

    e_t = jnp.swapaxes(embedded_adjancy_matrix, 2, 3)
    y = _make_esum(b_g, n, h_dim, chunk_rows=2)(e_t)
    y = y.reshape(b_g, n, h_dim * _LANES)

    wfc16 = jnp.repeat(Wf[:, 2 * h_dim:, :], _LANES, axis=1).astype(_BF16)
    wfab16 = Wf[:, :2 * h_dim, :].astype(_BF16)
    wih16 = Wih.astype(_BF16)
    whh16 = Whh.astype(_BF16)

    out = pl.pallas_call(
        _gru_kernel,
        grid=(b_g,),
        in_specs=[
            pl.BlockSpec((1, n, h_dim * _LANES), lambda b: (b, 0, 0)),
            pl.BlockSpec((1, n, h_dim), lambda b: (b, 0, 0)),
            pl.BlockSpec((r_rounds, 2 * h_dim, 6 * h_dim), lambda b: (0, 0, 0)),
            pl.BlockSpec((r_rounds, h_dim * _LANES, 6 * h_dim), lambda b: (0, 0, 0)),
            pl.BlockSpec((r_rounds, 6 * h_dim), lambda b: (0, 0)),
            pl.BlockSpec((r_rounds, 6 * h_dim, 3 * h_dim), lambda b: (0, 0, 0)),
            pl.BlockSpec((r_rounds, h_dim, 3 * h_dim), lambda b: (0, 0, 0)),
            pl.BlockSpec((r_rounds, 3 * h_dim), lambda b: (0, 0)),
            pl.BlockSpec((r_rounds, 3 * h_dim), lambda b: (0, 0)),
        ],
        out_specs=pl.BlockSpec((1, 1, h_dim), lambda b: (b, 0, 0)),
        out_shape=jax.ShapeDtypeStruct((b_g, 1, h_dim), jnp.float32),
        compiler_params=pltpu.CompilerParams(
            dimension_semantics=("arbitrary",),
        ),
    )(y, embedded_node, wfab16, wfc16, bf, wih16, whh16, bih, bhh)
    return out.reshape(b_g, h_dim)


# speedup vs baseline: 25.9609x; 1.1511x over previous
import jax
import jax.numpy as jnp
from jax import lax
from jax.experimental import pallas as pl
from jax.experimental.pallas import tpu as pltpu
from jax.experimental.pallas import tpu_sc as plsc

_F32 = jnp.float32
_BF16 = jnp.bfloat16
_I32 = jnp.int32

_NC = 2
_NS = 16
_NW = _NC * _NS
_LANES = 16


def _round_bf16(v):
    u = lax.bitcast_convert_type(v, _I32)
    lsb = jnp.bitwise_and(jnp.right_shift(u, 16), 1)
    r = jnp.bitwise_and(u + 32767 + lsb, _I32(-65536))
    return lax.bitcast_convert_type(r, _F32)


def _make_esum(b_g, n, h_dim, chunk_rows):
    rows = b_g * n
    rows_per_w = rows // _NW
    n_chunks = rows_per_w // chunk_rows
    jvecs = n // _LANES
    assert h_dim == 4 * _LANES and rows_per_w * _NW == rows
    assert n % rows_per_w == 0
    assert n_chunks * chunk_rows == rows_per_w and n % _LANES == 0

    def body(e_hbm, out_hbm, buf0, buf1, outbuf, sem0, sem1):
        wid = lax.axis_index("s") * _NC + lax.axis_index("c")
        b = wid // (n // rows_per_w)
        i0 = (wid % (n // rows_per_w)) * rows_per_w
        bufs = (buf0, buf1)
        sems = (sem0, sem1)

        def start(c):
            return pltpu.async_copy(
                e_hbm.at[b, pl.ds(i0 + c * chunk_rows, chunk_rows)],
                bufs[c % 2], sems[c % 2])

        cp = start(0)
        for c in range(n_chunks):
            nxt = start(c + 1) if c + 1 < n_chunks else None
            cp.wait()
            buf = bufs[c % 2]
            for r in range(chunk_rows):
                ob = (c * chunk_rows + r) * h_dim * _LANES

                def hbody(h, carry, buf=buf, r=r, ob=ob):
                    p = [_round_bf16(buf[r, h, pl.ds(g * _LANES, _LANES)])
                         for g in range(jvecs)]
                    while len(p) > 1:
                        p = [a + b for a, b in zip(p[::2], p[1::2])]
                    outbuf[pl.ds(ob + h * _LANES, _LANES)] = p[0]
                    return carry

                lax.fori_loop(0, h_dim, hbody, 0)
            cp = nxt
        out_words = rows_per_w * h_dim * _LANES
        pltpu.sync_copy(outbuf, out_hbm.at[pl.ds(wid * out_words, out_words)])

    return pl.kernel(
        body,
        mesh=plsc.VectorSubcoreMesh(core_axis_name="c", subcore_axis_name="s"),
        out_type=jax.ShapeDtypeStruct((rows * h_dim * _LANES,), _F32),
        scratch_types=[
            pltpu.VMEM((chunk_rows, h_dim, n), _F32),
            pltpu.VMEM((chunk_rows, h_dim, n), _F32),
            pltpu.VMEM((rows_per_w * h_dim * _LANES,), _F32),
            pltpu.SemaphoreType.DMA,
            pltpu.SemaphoreType.DMA,
        ],
    )


def _dot(a, b, precision=None):
    return lax.dot_general(
        a, b, (((1,), (0,)), ((), ())),
        precision=precision,
        preferred_element_type=_F32,
    )


def _gru_kernel(y_ref, node_ref, wfab_ref, wfc_ref, bf_ref, wih_ref,
                whh_ref, bih_ref, bhh_ref, out_ref):
    n = node_ref.shape[1]
    h_dim = node_ref.shape[2]
    r_rounds = wfab_ref.shape[0]
    hi = lax.Precision.HIGHEST

    yp = y_ref[0]
    pidx = lax.broadcasted_iota(jnp.int32, (yp.shape[1], h_dim), 0)
    hidx = lax.broadcasted_iota(jnp.int32, (yp.shape[1], h_dim), 1)
    sel = (pidx // _LANES == hidx).astype(_F32)
    y = _dot(yp, sel, lax.Precision.HIGHEST)
    h = node_ref[0]
    fn = _F32(n)
    for t in range(r_rounds):
        h16 = h.astype(_BF16)
        hterm = _dot(h16, wfab_ref[t, :h_dim, :]) * fn
        hs = jnp.sum(h16.astype(_F32), axis=0, keepdims=True)
        hsterm = _dot(hs, wfab_ref[t, h_dim:, :].astype(_F32), hi)
        eterm = _dot(y, wfc_ref[t].astype(_F32), hi)
        agg = (hterm + jnp.broadcast_to(hsterm, (n, 6 * h_dim)) + eterm
               + fn * bf_ref[t][None, :])
        gi = _dot(agg.astype(_BF16), wih_ref[t]) + bih_ref[t][None, :]
        gh = _dot(h16, whh_ref[t]) + bhh_ref[t][None, :]
        i_r, i_z, i_n = jnp.split(gi, 3, axis=-1)
        h_r, h_z, h_n = jnp.split(gh, 3, axis=-1)
        r = jax.nn.sigmoid(i_r + h_r)
        z = jax.nn.sigmoid(i_z + h_z)
        nn = jnp.tanh(i_n + r * h_n)
        h = (1.0 - z) * nn + z * h

    out_ref[...] = jnp.sum(h, axis=0, keepdims=True)[None]


def kernel(embedded_node, embedded_adjancy_matrix, Wf, bf, Wih, Whh, bih, bhh):
    b_g, n, _, h_dim = embedded_adjancy_matrix.shape
    r_rounds = Wf.shape[0]
    rows = b_g * n

    e_t = jnp.swapaxes(embedded_adjancy_matrix, 2, 3)
    y = _make_esum(b_g, n, h_dim, chunk_rows=2)(e_t)
    y = y.reshape(b_g, n, h_dim * _LANES)

    wfc16 = Wf[:, 2 * h_dim:, :].astype(_BF16)
    wfab16 = Wf[:, :2 * h_dim, :].astype(_BF16)
    wih16 = Wih.astype(_BF16)
    whh16 = Whh.astype(_BF16)

    out = pl.pallas_call(
        _gru_kernel,
        grid=(b_g,),
        in_specs=[
            pl.BlockSpec((1, n, h_dim * _LANES), lambda b: (b, 0, 0)),
            pl.BlockSpec((1, n, h_dim), lambda b: (b, 0, 0)),
            pl.BlockSpec((r_rounds, 2 * h_dim, 6 * h_dim), lambda b: (0, 0, 0)),
            pl.BlockSpec((r_rounds, h_dim, 6 * h_dim), lambda b: (0, 0, 0)),
            pl.BlockSpec((r_rounds, 6 * h_dim), lambda b: (0, 0)),
            pl.BlockSpec((r_rounds, 6 * h_dim, 3 * h_dim), lambda b: (0, 0, 0)),
            pl.BlockSpec((r_rounds, h_dim, 3 * h_dim), lambda b: (0, 0, 0)),
            pl.BlockSpec((r_rounds, 3 * h_dim), lambda b: (0, 0)),
            pl.BlockSpec((r_rounds, 3 * h_dim), lambda b: (0, 0)),
        ],
        out_specs=pl.BlockSpec((1, 1, h_dim), lambda b: (b, 0, 0)),
        out_shape=jax.ShapeDtypeStruct((b_g, 1, h_dim), jnp.float32),
        compiler_params=pltpu.CompilerParams(
            dimension_semantics=("arbitrary",),
        ),
    )(y, embedded_node, wfab16, wfc16, bf, wih16, whh16, bih, bhh)
    return out.reshape(b_g, h_dim)


# speedup vs baseline: 26.9317x; 1.0374x over previous
import jax
import jax.numpy as jnp
from jax import lax
from jax.experimental import pallas as pl
from jax.experimental.pallas import tpu as pltpu
from jax.experimental.pallas import tpu_sc as plsc

_F32 = jnp.float32
_BF16 = jnp.bfloat16
_I32 = jnp.int32

_NC = 2
_NS = 16
_NW = _NC * _NS
_LANES = 16


def _round_bf16(v):
    u = lax.bitcast_convert_type(v, _I32)
    lsb = jnp.bitwise_and(jnp.right_shift(u, 16), 1)
    r = jnp.bitwise_and(u + 32767 + lsb, _I32(-65536))
    return lax.bitcast_convert_type(r, _F32)


def _make_esum(b_g, n, h_dim, chunk_rows):
    rows = b_g * n
    rows_per_w = rows // _NW
    n_chunks = rows_per_w // chunk_rows
    jvecs = n // _LANES
    assert h_dim == 4 * _LANES and rows_per_w * _NW == rows
    assert n % rows_per_w == 0
    assert n_chunks * chunk_rows == rows_per_w and n % _LANES == 0

    def body(e_hbm, out_hbm, buf0, buf1, outbuf, sem0, sem1):
        wid = lax.axis_index("s") * _NC + lax.axis_index("c")
        b = wid // (n // rows_per_w)
        i0 = (wid % (n // rows_per_w)) * rows_per_w
        bufs = (buf0, buf1)
        sems = (sem0, sem1)

        def start(c):
            return pltpu.async_copy(
                e_hbm.at[b, pl.ds(i0 + c * chunk_rows, chunk_rows)],
                bufs[c % 2], sems[c % 2])

        cp = start(0)
        for c in range(n_chunks):
            nxt = start(c + 1) if c + 1 < n_chunks else None
            cp.wait()
            buf = bufs[c % 2]
            for r in range(chunk_rows):
                ob = (c * chunk_rows + r) * h_dim * _LANES

                def hbody(h, carry, buf=buf, r=r, ob=ob):
                    p = [_round_bf16(buf[r, h, pl.ds(g * _LANES, _LANES)])
                         for g in range(jvecs)]
                    while len(p) > 1:
                        p = [a + b for a, b in zip(p[::2], p[1::2])]
                    outbuf[pl.ds(ob + h * _LANES, _LANES)] = p[0]
                    return carry

                lax.fori_loop(0, h_dim, hbody, 0)
            cp = nxt
        out_words = rows_per_w * h_dim * _LANES
        pltpu.sync_copy(outbuf, out_hbm.at[pl.ds(wid * out_words, out_words)])

    return pl.kernel(
        body,
        mesh=plsc.VectorSubcoreMesh(core_axis_name="c", subcore_axis_name="s"),
        out_type=jax.ShapeDtypeStruct((rows * h_dim * _LANES,), _F32),
        scratch_types=[
            pltpu.VMEM((chunk_rows, h_dim, n), _F32),
            pltpu.VMEM((chunk_rows, h_dim, n), _F32),
            pltpu.VMEM((rows_per_w * h_dim * _LANES,), _F32),
            pltpu.SemaphoreType.DMA,
            pltpu.SemaphoreType.DMA,
        ],
    )


def _dot(a, b, precision=None):
    return lax.dot_general(
        a, b, (((1,), (0,)), ((), ())),
        precision=precision,
        preferred_element_type=_F32,
    )


def _gru_kernel(y_ref, node_ref, wfab_ref, wfc_ref, bf_ref, wih_ref,
                whh_ref, bih_ref, bhh_ref, out_ref):
    b_g, n, h_dim = node_ref.shape
    rows = b_g * n
    r_rounds = wfab_ref.shape[0]
    hi = lax.Precision.HIGHEST

    yp = y_ref[...].reshape(rows, _LANES * h_dim)
    pidx = lax.broadcasted_iota(jnp.int32, (yp.shape[1], h_dim), 0)
    hidx = lax.broadcasted_iota(jnp.int32, (yp.shape[1], h_dim), 1)
    sel = (pidx // _LANES == hidx).astype(_F32)
    y = _dot(yp, sel, lax.Precision.HIGHEST)
    h = node_ref[...].reshape(rows, h_dim)
    fn = _F32(n)
    for t in range(r_rounds):
        h16 = h.astype(_BF16)
        hterm = _dot(h16, wfab_ref[t, :h_dim, :]) * fn
        hs = jnp.sum(h16.astype(_F32).reshape(b_g, n, h_dim), axis=1)
        hsterm = _dot(hs, wfab_ref[t, h_dim:, :].astype(_F32), hi)
        hsrows = jnp.broadcast_to(hsterm[:, None, :], (b_g, n, 6 * h_dim))
        eterm = _dot(y, wfc_ref[t].astype(_F32), hi)
        agg = (hterm + hsrows.reshape(rows, 6 * h_dim) + eterm
               + fn * bf_ref[t][None, :])
        gi = _dot(agg.astype(_BF16), wih_ref[t]) + bih_ref[t][None, :]
        gh = _dot(h16, whh_ref[t]) + bhh_ref[t][None, :]
        i_r, i_z, i_n = jnp.split(gi, 3, axis=-1)
        h_r, h_z, h_n = jnp.split(gh, 3, axis=-1)
        r = jax.nn.sigmoid(i_r + h_r)
        z = jax.nn.sigmoid(i_z + h_z)
        nn = jnp.tanh(i_n + r * h_n)
        h = (1.0 - z) * nn + z * h

    out_ref[...] = jnp.sum(h.reshape(b_g, n, h_dim), axis=1)


def kernel(embedded_node, embedded_adjancy_matrix, Wf, bf, Wih, Whh, bih, bhh):
    b_g, n, _, h_dim = embedded_adjancy_matrix.shape
    r_rounds = Wf.shape[0]
    rows = b_g * n

    e_t = jnp.swapaxes(embedded_adjancy_matrix, 2, 3)
    y = _make_esum(b_g, n, h_dim, chunk_rows=2)(e_t)
    y = y.reshape(b_g, n, h_dim * _LANES)

    wfc16 = Wf[:, 2 * h_dim:, :].astype(_BF16)
    wfab16 = Wf[:, :2 * h_dim, :].astype(_BF16)
    wih16 = Wih.astype(_BF16)
    whh16 = Whh.astype(_BF16)

    out = pl.pallas_call(
        _gru_kernel,
        out_shape=jax.ShapeDtypeStruct((b_g, h_dim), jnp.float32),
    )(y, embedded_node, wfab16, wfc16, bf, wih16, whh16, bih, bhh)
    return out


# speedup vs baseline: 28.9475x; 1.0748x over previous
import jax
import jax.numpy as jnp
from jax import lax
from jax.experimental import pallas as pl
from jax.experimental.pallas import tpu as pltpu
from jax.experimental.pallas import tpu_sc as plsc

_F32 = jnp.float32
_BF16 = jnp.bfloat16
_I32 = jnp.int32

_NC = 2
_NS = 16
_NW = _NC * _NS
_LANES = 16


def _round_bf16(v):
    u = lax.bitcast_convert_type(v, _I32)
    lsb = jnp.bitwise_and(jnp.right_shift(u, 16), 1)
    r = jnp.bitwise_and(u + 32767 + lsb, _I32(-65536))
    return lax.bitcast_convert_type(r, _F32)


def _make_esum(b_g, n, h_dim, chunk_rows):
    rows = b_g * n
    rows_per_w = rows // _NW
    n_chunks = rows_per_w // chunk_rows
    jvecs = n // _LANES
    assert h_dim == 4 * _LANES and rows_per_w * _NW == rows
    assert n % rows_per_w == 0
    assert n_chunks * chunk_rows == rows_per_w and n % _LANES == 0

    def body(e_hbm, out_hbm, buf0, buf1, outbuf, sem0, sem1):
        wid = lax.axis_index("s") * _NC + lax.axis_index("c")
        b = wid // (n // rows_per_w)
        i0 = (wid % (n // rows_per_w)) * rows_per_w
        bufs = (buf0, buf1)
        sems = (sem0, sem1)

        def start(c):
            return pltpu.async_copy(
                e_hbm.at[b, pl.ds(i0 + c * chunk_rows, chunk_rows)],
                bufs[c % 2], sems[c % 2])

        cp = start(0)
        for c in range(n_chunks):
            nxt = start(c + 1) if c + 1 < n_chunks else None
            cp.wait()
            buf = bufs[c % 2]
            for r in range(chunk_rows):
                orow = c * chunk_rows + r

                def hbody(h, carry, buf=buf, r=r, orow=orow):
                    p = [_round_bf16(buf[r, h, pl.ds(g * _LANES, _LANES)])
                         for g in range(jvecs)]
                    while len(p) > 1:
                        p = [a + b for a, b in zip(p[::2], p[1::2])]
                    outbuf[orow, pl.ds(h * _LANES, _LANES)] = p[0]
                    return carry

                lax.fori_loop(0, h_dim, hbody, 0)
            cp = nxt
        pltpu.sync_copy(
            outbuf, out_hbm.at[pl.ds(wid * rows_per_w, rows_per_w)])

    return pl.kernel(
        body,
        mesh=plsc.VectorSubcoreMesh(core_axis_name="c", subcore_axis_name="s"),
        out_type=jax.ShapeDtypeStruct((rows, h_dim * _LANES), _F32),
        scratch_types=[
            pltpu.VMEM((chunk_rows, h_dim, n), _F32),
            pltpu.VMEM((chunk_rows, h_dim, n), _F32),
            pltpu.VMEM((rows_per_w, h_dim * _LANES), _F32),
            pltpu.SemaphoreType.DMA,
            pltpu.SemaphoreType.DMA,
        ],
    )


def _dot(a, b, precision=None):
    return lax.dot_general(
        a, b, (((1,), (0,)), ((), ())),
        precision=precision,
        preferred_element_type=_F32,
    )


def _gru_kernel(y_ref, node_ref, wfab_ref, wfc_ref, bf_ref, wih_ref,
                whh_ref, bih_ref, bhh_ref, out_ref):
    b_g, n, h_dim = node_ref.shape
    rows = b_g * n
    r_rounds = wfab_ref.shape[0]
    hi = lax.Precision.HIGHEST

    yp = y_ref[...]
    pidx = lax.broadcasted_iota(jnp.int32, (yp.shape[1], h_dim), 0)
    hidx = lax.broadcasted_iota(jnp.int32, (yp.shape[1], h_dim), 1)
    sel = (pidx // _LANES == hidx).astype(_F32)
    y = _dot(yp, sel, lax.Precision.HIGHEST)
    h = node_ref[...].reshape(rows, h_dim)
    fn = _F32(n)
    for t in range(r_rounds):
        h16 = h.astype(_BF16)
        hterm = _dot(h16, wfab_ref[t, :h_dim, :]) * fn
        hs = jnp.sum(h16.astype(_F32).reshape(b_g, n, h_dim), axis=1)
        hsterm = _dot(hs, wfab_ref[t, h_dim:, :].astype(_F32), hi)
        hsrows = jnp.broadcast_to(hsterm[:, None, :], (b_g, n, 6 * h_dim))
        eterm = _dot(y, wfc_ref[t].astype(_F32), hi)
        agg = (hterm + hsrows.reshape(rows, 6 * h_dim) + eterm
               + fn * bf_ref[t][None, :])
        gi = _dot(agg.astype(_BF16), wih_ref[t]) + bih_ref[t][None, :]
        gh = _dot(h16, whh_ref[t]) + bhh_ref[t][None, :]
        i_r, i_z, i_n = jnp.split(gi, 3, axis=-1)
        h_r, h_z, h_n = jnp.split(gh, 3, axis=-1)
        r = jax.nn.sigmoid(i_r + h_r)
        z = jax.nn.sigmoid(i_z + h_z)
        nn = jnp.tanh(i_n + r * h_n)
        h = (1.0 - z) * nn + z * h

    out_ref[...] = jnp.sum(h.reshape(b_g, n, h_dim), axis=1)


def kernel(embedded_node, embedded_adjancy_matrix, Wf, bf, Wih, Whh, bih, bhh):
    b_g, n, _, h_dim = embedded_adjancy_matrix.shape
    r_rounds = Wf.shape[0]
    rows = b_g * n

    e_t = jnp.swapaxes(embedded_adjancy_matrix, 2, 3)
    y = _make_esum(b_g, n, h_dim, chunk_rows=2)(e_t)

    wfc16 = Wf[:, 2 * h_dim:, :].astype(_BF16)
    wfab16 = Wf[:, :2 * h_dim, :].astype(_BF16)
    wih16 = Wih.astype(_BF16)
    whh16 = Whh.astype(_BF16)

    out = pl.pallas_call(
        _gru_kernel,
        out_shape=jax.ShapeDtypeStruct((b_g, h_dim), jnp.float32),
    )(y, embedded_node, wfab16, wfc16, bf, wih16, whh16, bih, bhh)
    return out
